# baseline (device time: 107587 ns/iter reference)
import jax
import jax.numpy as jnp
from jax import lax
from jax.experimental import pallas as pl
from jax.experimental.pallas import tpu as pltpu

N_DEV = 4


def kernel(O, Wo):
    B, S, H, D = O.shape
    HD = H * D
    N = Wo.shape[1]
    Nh = N // 2
    s_chunk = S // N_DEV
    SEG = B

    O_f = O.reshape(B * S, H, D)

    def body(o_ref, wo_ref, out_ref,
             o_bf, wo_bf, comm_r, comm_l, send_r, recv_r, send_l, recv_l):
        my = lax.axis_index("i")
        left = lax.rem(my + (N_DEV - 1), N_DEV)
        right = lax.rem(my + 1, N_DEV)

        barrier_sem = pltpu.get_barrier_semaphore()
        for nbr in (left, right):
            pl.semaphore_signal(
                barrier_sem, inc=1,
                device_id=(nbr,), device_id_type=pl.DeviceIdType.MESH,
            )
        wo_bf[...] = wo_ref[...].astype(jnp.bfloat16)
        pl.semaphore_wait(barrier_sem, 2)

        def relayout_batch(b):
            for h in range(H):
                o_bf[pl.ds(b * S, S), h * D:(h + 1) * D] = (
                    o_ref[pl.ds(b * S, S), h, :].astype(jnp.bfloat16)
                )

        def partial(c, b, col0):
            x = o_bf[pl.ds(b * S + c * s_chunk, s_chunk), :]
            w = wo_bf[:, col0:col0 + Nh]
            return jnp.dot(x, w, preferred_element_type=jnp.float32)

        def make_rdma(comm, send, recv, h, s, dev):
            return pltpu.make_async_remote_copy(
                src_ref=comm.at[h, s], dst_ref=comm.at[h + 1, s],
                send_sem=send.at[h, s], recv_sem=recv.at[h, s],
                device_id=(dev,), device_id_type=pl.DeviceIdType.MESH,
            )

        in_flight = []

        def start(comm, send, recv, h, s, dev):
            rdma = make_rdma(comm, send, recv, h, s, dev)
            rdma.start()
            in_flight.append(rdma)

        rcv_r = [lax.rem(my + 2 - h, N_DEV) for h in range(N_DEV - 1)]
        rcv_l = [lax.rem(my + 2 + h, N_DEV) for h in range(N_DEV - 1)]

        c_r0 = lax.rem(my + 3, N_DEV)
        c_l0 = lax.rem(my + 1, N_DEV)
        for s in range(SEG):
            relayout_batch(s)
            comm_r[0, s] = partial(c_r0, s, 0).astype(jnp.bfloat16)
            start(comm_r, send_r, recv_r, 0, s, right)
            comm_l[0, s] = partial(c_l0, s, Nh).astype(jnp.bfloat16)
            start(comm_l, send_l, recv_l, 0, s, left)

        for h in range(N_DEV - 2):
            for s in range(SEG):
                nxt_r = partial(rcv_r[h], s, 0)
                make_rdma(comm_r, send_r, recv_r, h, s, right).wait_recv()
                comm_r[h + 1, s] = (
                    comm_r[h + 1, s].astype(jnp.float32) + nxt_r
                ).astype(jnp.bfloat16)
                start(comm_r, send_r, recv_r, h + 1, s, right)

                nxt_l = partial(rcv_l[h], s, Nh)
                make_rdma(comm_l, send_l, recv_l, h, s, left).wait_recv()
                comm_l[h + 1, s] = (
                    comm_l[h + 1, s].astype(jnp.float32) + nxt_l
                ).astype(jnp.bfloat16)
                start(comm_l, send_l, recv_l, h + 1, s, left)

        hf = N_DEV - 2
        for s in range(SEG):
            nxt_r = partial(my, s, 0)
            make_rdma(comm_r, send_r, recv_r, hf, s, right).wait_recv()
            out_ref[s, :, 0:Nh] = comm_r[hf + 1, s].astype(jnp.float32) + nxt_r
            nxt_l = partial(my, s, Nh)
            make_rdma(comm_l, send_l, recv_l, hf, s, left).wait_recv()
            out_ref[s, :, Nh:N] = comm_l[hf + 1, s].astype(jnp.float32) + nxt_l

        for rdma in in_flight:
            rdma.wait_send()

    return pl.pallas_call(
        body,
        out_shape=jax.ShapeDtypeStruct((B, s_chunk, N), jnp.float32),
        in_specs=[
            pl.BlockSpec(memory_space=pltpu.VMEM),
            pl.BlockSpec(memory_space=pltpu.VMEM),
        ],
        out_specs=pl.BlockSpec(memory_space=pltpu.VMEM),
        scratch_shapes=[
            pltpu.VMEM((B * S, HD), jnp.bfloat16),
            pltpu.VMEM((HD, N), jnp.bfloat16),
            pltpu.VMEM((N_DEV, SEG, s_chunk, Nh), jnp.bfloat16),
            pltpu.VMEM((N_DEV, SEG, s_chunk, Nh), jnp.bfloat16),
            pltpu.SemaphoreType.DMA((N_DEV - 1, SEG)),
            pltpu.SemaphoreType.DMA((N_DEV - 1, SEG)),
            pltpu.SemaphoreType.DMA((N_DEV - 1, SEG)),
            pltpu.SemaphoreType.DMA((N_DEV - 1, SEG)),
        ],
        compiler_params=pltpu.CompilerParams(
            collective_id=0, vmem_limit_bytes=100 * 1024 * 1024
        ),
    )(O_f, Wo)


# device time: 104087 ns/iter; 1.0336x vs baseline; 1.0336x over previous
import jax
import jax.numpy as jnp
from jax import lax
from jax.experimental import pallas as pl
from jax.experimental.pallas import tpu as pltpu

N_DEV = 4


def kernel(O, Wo):
    B, S, H, D = O.shape
    HD = H * D
    N = Wo.shape[1]
    Nh = N // 2
    s_chunk = S // N_DEV
    SEG = B

    O_f = O.reshape(B * S, HD)

    def body(o_ref, wo_ref, out_ref,
             wo_bf, comm_r, comm_l, send_r, recv_r, send_l, recv_l):
        my = lax.axis_index("i")
        left = lax.rem(my + (N_DEV - 1), N_DEV)
        right = lax.rem(my + 1, N_DEV)

        barrier_sem = pltpu.get_barrier_semaphore()
        for nbr in (left, right):
            pl.semaphore_signal(
                barrier_sem, inc=1,
                device_id=(nbr,), device_id_type=pl.DeviceIdType.MESH,
            )
        wo_bf[...] = wo_ref[...].astype(jnp.bfloat16)
        pl.semaphore_wait(barrier_sem, 2)

        def partial(c, b, col0):
            x = o_ref[pl.ds(b * S + c * s_chunk, s_chunk), :]
            w = wo_bf[:, col0:col0 + Nh]
            return jnp.dot(
                x.astype(jnp.bfloat16), w, preferred_element_type=jnp.float32
            )

        def make_rdma(comm, send, recv, h, s, dev):
            return pltpu.make_async_remote_copy(
                src_ref=comm.at[h, s], dst_ref=comm.at[h + 1, s],
                send_sem=send.at[h, s], recv_sem=recv.at[h, s],
                device_id=(dev,), device_id_type=pl.DeviceIdType.MESH,
            )

        in_flight = []

        def start(comm, send, recv, h, s, dev):
            rdma = make_rdma(comm, send, recv, h, s, dev)
            rdma.start()
            in_flight.append(rdma)

        rcv_r = [lax.rem(my + 2 - h, N_DEV) for h in range(N_DEV - 1)]
        rcv_l = [lax.rem(my + 2 + h, N_DEV) for h in range(N_DEV - 1)]

        c_r0 = lax.rem(my + 3, N_DEV)
        c_l0 = lax.rem(my + 1, N_DEV)
        for s in range(SEG):
            comm_r[0, s] = partial(c_r0, s, 0).astype(jnp.bfloat16)
            start(comm_r, send_r, recv_r, 0, s, right)
            comm_l[0, s] = partial(c_l0, s, Nh).astype(jnp.bfloat16)
            start(comm_l, send_l, recv_l, 0, s, left)

        for h in range(N_DEV - 2):
            for s in range(SEG):
                nxt_r = partial(rcv_r[h], s, 0)
                make_rdma(comm_r, send_r, recv_r, h, s, right).wait_recv()
                comm_r[h + 1, s] = (
                    comm_r[h + 1, s].astype(jnp.float32) + nxt_r
                ).astype(jnp.bfloat16)
                start(comm_r, send_r, recv_r, h + 1, s, right)

                nxt_l = partial(rcv_l[h], s, Nh)
                make_rdma(comm_l, send_l, recv_l, h, s, left).wait_recv()
                comm_l[h + 1, s] = (
                    comm_l[h + 1, s].astype(jnp.float32) + nxt_l
                ).astype(jnp.bfloat16)
                start(comm_l, send_l, recv_l, h + 1, s, left)

        hf = N_DEV - 2
        for s in range(SEG):
            nxt_r = partial(my, s, 0)
            make_rdma(comm_r, send_r, recv_r, hf, s, right).wait_recv()
            out_ref[s, :, 0:Nh] = comm_r[hf + 1, s].astype(jnp.float32) + nxt_r
            nxt_l = partial(my, s, Nh)
            make_rdma(comm_l, send_l, recv_l, hf, s, left).wait_recv()
            out_ref[s, :, Nh:N] = comm_l[hf + 1, s].astype(jnp.float32) + nxt_l

        for rdma in in_flight:
            rdma.wait_send()

    return pl.pallas_call(
        body,
        out_shape=jax.ShapeDtypeStruct((B, s_chunk, N), jnp.float32),
        in_specs=[
            pl.BlockSpec(memory_space=pltpu.VMEM),
            pl.BlockSpec(memory_space=pltpu.VMEM),
        ],
        out_specs=pl.BlockSpec(memory_space=pltpu.VMEM),
        scratch_shapes=[
            pltpu.VMEM((HD, N), jnp.bfloat16),
            pltpu.VMEM((N_DEV, SEG, s_chunk, Nh), jnp.bfloat16),
            pltpu.VMEM((N_DEV, SEG, s_chunk, Nh), jnp.bfloat16),
            pltpu.SemaphoreType.DMA((N_DEV - 1, SEG)),
            pltpu.SemaphoreType.DMA((N_DEV - 1, SEG)),
            pltpu.SemaphoreType.DMA((N_DEV - 1, SEG)),
            pltpu.SemaphoreType.DMA((N_DEV - 1, SEG)),
        ],
        compiler_params=pltpu.CompilerParams(
            collective_id=0, vmem_limit_bytes=100 * 1024 * 1024
        ),
    )(O_f, Wo)


# device time: 82720 ns/iter; 1.3006x vs baseline; 1.2583x over previous
import jax
import jax.numpy as jnp
from jax import lax
from jax.experimental import pallas as pl
from jax.experimental.pallas import tpu as pltpu

N_DEV = 4


def kernel(O, Wo):
    B, S, H, D = O.shape
    HD = H * D
    N = Wo.shape[1]
    Nh = N // 2
    s_chunk = S // N_DEV
    SEG = B

    O_t = O.transpose(0, 2, 3, 1)

    def body(o_ref, wo_ref, out_ref,
             wo_bf, comm_r, comm_l, send_r, recv_r, send_l, recv_l):
        my = lax.axis_index("i")
        left = lax.rem(my + (N_DEV - 1), N_DEV)
        right = lax.rem(my + 1, N_DEV)

        barrier_sem = pltpu.get_barrier_semaphore()
        for nbr in (left, right):
            pl.semaphore_signal(
                barrier_sem, inc=1,
                device_id=(nbr,), device_id_type=pl.DeviceIdType.MESH,
            )
        wo_bf[...] = wo_ref[...].astype(jnp.bfloat16)
        pl.semaphore_wait(barrier_sem, 2)

        def partial(c, b, col0):
            xT = o_ref[b, :, :, pl.ds(c * s_chunk, s_chunk)]
            xT = xT.reshape(HD, s_chunk)
            w = wo_bf[:, col0:col0 + Nh]
            return jax.lax.dot_general(
                xT.astype(jnp.bfloat16), w,
                dimension_numbers=(((0,), (0,)), ((), ())),
                preferred_element_type=jnp.float32,
            )

        def make_rdma(comm, send, recv, h, s, dev):
            return pltpu.make_async_remote_copy(
                src_ref=comm.at[h, s], dst_ref=comm.at[h + 1, s],
                send_sem=send.at[h, s], recv_sem=recv.at[h, s],
                device_id=(dev,), device_id_type=pl.DeviceIdType.MESH,
            )

        in_flight = []

        def start(comm, send, recv, h, s, dev):
            rdma = make_rdma(comm, send, recv, h, s, dev)
            rdma.start()
            in_flight.append(rdma)

        rcv_r = [lax.rem(my + 2 - h, N_DEV) for h in range(N_DEV - 1)]
        rcv_l = [lax.rem(my + 2 + h, N_DEV) for h in range(N_DEV - 1)]

        c_r0 = lax.rem(my + 3, N_DEV)
        c_l0 = lax.rem(my + 1, N_DEV)
        for s in range(SEG):
            comm_r[0, s] = partial(c_r0, s, 0).astype(jnp.bfloat16)
            start(comm_r, send_r, recv_r, 0, s, right)
            comm_l[0, s] = partial(c_l0, s, Nh).astype(jnp.bfloat16)
            start(comm_l, send_l, recv_l, 0, s, left)

        for h in range(N_DEV - 2):
            for s in range(SEG):
                nxt_r = partial(rcv_r[h], s, 0)
                make_rdma(comm_r, send_r, recv_r, h, s, right).wait_recv()
                comm_r[h + 1, s] = (
                    comm_r[h + 1, s].astype(jnp.float32) + nxt_r
                ).astype(jnp.bfloat16)
                start(comm_r, send_r, recv_r, h + 1, s, right)

                nxt_l = partial(rcv_l[h], s, Nh)
                make_rdma(comm_l, send_l, recv_l, h, s, left).wait_recv()
                comm_l[h + 1, s] = (
                    comm_l[h + 1, s].astype(jnp.float32) + nxt_l
                ).astype(jnp.bfloat16)
                start(comm_l, send_l, recv_l, h + 1, s, left)

        hf = N_DEV - 2
        for s in range(SEG):
            nxt_r = partial(my, s, 0)
            make_rdma(comm_r, send_r, recv_r, hf, s, right).wait_recv()
            out_ref[s, :, 0:Nh] = comm_r[hf + 1, s].astype(jnp.float32) + nxt_r
            nxt_l = partial(my, s, Nh)
            make_rdma(comm_l, send_l, recv_l, hf, s, left).wait_recv()
            out_ref[s, :, Nh:N] = comm_l[hf + 1, s].astype(jnp.float32) + nxt_l

        for rdma in in_flight:
            rdma.wait_send()

    return pl.pallas_call(
        body,
        out_shape=jax.ShapeDtypeStruct((B, s_chunk, N), jnp.float32),
        in_specs=[
            pl.BlockSpec(memory_space=pltpu.VMEM),
            pl.BlockSpec(memory_space=pltpu.VMEM),
        ],
        out_specs=pl.BlockSpec(memory_space=pltpu.VMEM),
        scratch_shapes=[
            pltpu.VMEM((HD, N), jnp.bfloat16),
            pltpu.VMEM((N_DEV, SEG, s_chunk, Nh), jnp.bfloat16),
            pltpu.VMEM((N_DEV, SEG, s_chunk, Nh), jnp.bfloat16),
            pltpu.SemaphoreType.DMA((N_DEV - 1, SEG)),
            pltpu.SemaphoreType.DMA((N_DEV - 1, SEG)),
            pltpu.SemaphoreType.DMA((N_DEV - 1, SEG)),
            pltpu.SemaphoreType.DMA((N_DEV - 1, SEG)),
        ],
        compiler_params=pltpu.CompilerParams(collective_id=0),
    )(O_t, Wo)
